# baseline (speedup 1.0000x reference)
import functools

import jax
import jax.numpy as jnp
from jax import lax
from jax.experimental import pallas as pl
from jax.experimental.pallas import tpu as pltpu
from jax.experimental.pallas import tpu_sc as plsc

M = 16384
N = 8192
D = 256

BM = 512
BN = 1024
MB = M // BM
NB = N // BN

EPS = 1e-8

LANES = 128
NG = BN // LANES
RCH = 64

NC = 2
NS = 16
NW = NC * NS
ROWS_PER_W = M // NW
GCHUNK = 256


def _normalize_cb_body(c_ref, cb_ref, cbn_ref):
    c = c_ref[...]
    n1 = jnp.sqrt(jnp.sum(c * c, axis=1, keepdims=True))
    cb = c / (n1 + EPS)
    n2 = jnp.sqrt(jnp.sum(cb * cb, axis=1, keepdims=True))
    cbn = cb / (n2 + EPS)
    cb_ref[...] = cb
    cbn_ref[...] = cbn


def _normalize_cb(codebook):
    return pl.pallas_call(
        _normalize_cb_body,
        grid=(8,),
        in_specs=[pl.BlockSpec((N // 8, D), lambda i: (i, 0))],
        out_specs=[
            pl.BlockSpec((N // 8, D), lambda i: (i, 0)),
            pl.BlockSpec((N // 8, D), lambda i: (i, 0)),
        ],
        out_shape=[
            jax.ShapeDtypeStruct((N, D), jnp.float32),
            jax.ShapeDtypeStruct((N, D), jnp.float32),
        ],
    )(codebook)


def _normalize_x_body(x_ref, xn_ref):
    x = x_ref[...]
    nrm = jnp.sqrt(jnp.sum(x * x, axis=1, keepdims=True))
    xn_ref[...] = x / (nrm + EPS)


def _normalize_x(x):
    return pl.pallas_call(
        _normalize_x_body,
        grid=(16,),
        in_specs=[pl.BlockSpec((M // 16, D), lambda i: (i, 0))],
        out_specs=pl.BlockSpec((M // 16, D), lambda i: (i, 0)),
        out_shape=jax.ShapeDtypeStruct((M, D), jnp.float32),
    )(x)


def _argmax_body(xn_ref, cbn_ref, vmax_ref, vidx_ref):
    n = pl.program_id(1)

    @pl.when(n == 0)
    def _init():
        vmax_ref[...] = jnp.full((1, BM, LANES), -jnp.inf, jnp.float32)
        vidx_ref[...] = jnp.zeros((1, BM, LANES), jnp.int32)

    cbn = cbn_ref[...]
    for r in range(BM // RCH):
        rows = slice(r * RCH, (r + 1) * RCH)
        xr = xn_ref[rows, :]
        s = lax.dot_general(xr, cbn, (((1,), (1,)), ((), ())),
                            preferred_element_type=jnp.float32)
        v = vmax_ref[0, rows, :]
        ii = vidx_ref[0, rows, :]
        for g in range(NG):
            sg = s[:, g * LANES:(g + 1) * LANES]
            gt = sg > v
            v = jnp.maximum(sg, v)
            ii = jnp.where(gt, n * NG + g, ii)
        vmax_ref[0, rows, :] = v
        vidx_ref[0, rows, :] = ii


def _normalize_argmax(xn, cbn):
    vmax3, vidx3 = pl.pallas_call(
        _argmax_body,
        grid=(MB, NB),
        in_specs=[
            pl.BlockSpec((BM, D), lambda m, n: (m, 0)),
            pl.BlockSpec((BN, D), lambda m, n: (n, 0)),
        ],
        out_specs=[
            pl.BlockSpec((1, BM, LANES), lambda m, n: (m, 0, 0)),
            pl.BlockSpec((1, BM, LANES), lambda m, n: (m, 0, 0)),
        ],
        out_shape=[
            jax.ShapeDtypeStruct((MB, BM, LANES), jnp.float32),
            jax.ShapeDtypeStruct((MB, BM, LANES), jnp.int32),
        ],
    )(xn, cbn)
    return vmax3.reshape(M, LANES), vidx3.reshape(M, LANES)


FR = 2048


def _finish_body(v_ref, i_ref, idx_ref):
    v = v_ref[...]
    ii = i_ref[...]
    bmax = jnp.max(v, axis=1, keepdims=True)
    lane = lax.broadcasted_iota(jnp.int32, (FR, LANES), 1)
    idxfull = ii * LANES + lane
    bidx = jnp.min(jnp.where(v == bmax, idxfull, N), axis=1)
    idx_ref[...] = bidx.reshape(1, 1, FR)


def _finish(vmax, vidx):
    idx3 = pl.pallas_call(
        _finish_body,
        grid=(M // FR,),
        in_specs=[
            pl.BlockSpec((FR, LANES), lambda i: (i, 0)),
            pl.BlockSpec((FR, LANES), lambda i: (i, 0)),
        ],
        out_specs=pl.BlockSpec((1, 1, FR), lambda i: (i, 0, 0)),
        out_shape=jax.ShapeDtypeStruct((M // FR, 1, FR), jnp.int32),
    )(vmax, vidx)
    return idx3.reshape(M)


def _gather_body(cb_hbm, idx_hbm, z_hbm, idx_v, rows_v, sem):
    c = lax.axis_index("c")
    s = lax.axis_index("s")
    wid = s * NC + c
    base = wid * ROWS_PER_W
    for ch in range(ROWS_PER_W // GCHUNK):
        off = base + ch * GCHUNK
        pltpu.sync_copy(idx_hbm.at[pl.ds(off, GCHUNK)], idx_v)
        pltpu.async_copy(cb_hbm.at[idx_v], rows_v, sem).wait()
        pltpu.sync_copy(rows_v, z_hbm.at[pl.ds(off, GCHUNK)])


@functools.cache
def _sc_gather():
    return pl.kernel(
        _gather_body,
        out_type=jax.ShapeDtypeStruct((M, D), jnp.float32),
        mesh=plsc.VectorSubcoreMesh(core_axis_name="c", subcore_axis_name="s",
                                    num_cores=NC, num_subcores=NS),
        scratch_types=[
            pltpu.VMEM((GCHUNK,), jnp.int32),
            pltpu.VMEM((GCHUNK, D), jnp.float32),
            pltpu.SemaphoreType.DMA,
        ],
    )


def kernel(x, codebook, training=False):
    del training
    cb, cbn = _normalize_cb(codebook)
    xn = _normalize_x(x)
    vmax, vidx = _normalize_argmax(xn, cbn)
    indices = _finish(vmax, vidx)
    z = _sc_gather()(cb, indices)
    return (z, z, xn, indices)


# speedup vs baseline: 1.4045x; 1.4045x over previous
import functools

import jax
import jax.numpy as jnp
from jax import lax
from jax.experimental import pallas as pl
from jax.experimental.pallas import tpu as pltpu
from jax.experimental.pallas import tpu_sc as plsc

M = 16384
N = 8192
D = 256

BM = 512
BN = 1024
MB = M // BM
NB = N // BN

EPS = 1e-8

LANES = 128
NG = BN // LANES
RCH = 64

NC = 2
NS = 16
NW = NC * NS
ROWS_PER_W = M // NW
GCHUNK = 256


def _normalize_cb_body(c_ref, cb_ref, cbn_ref):
    c = c_ref[...]
    n1 = jnp.sqrt(jnp.sum(c * c, axis=1, keepdims=True))
    cb = c / (n1 + EPS)
    n2 = jnp.sqrt(jnp.sum(cb * cb, axis=1, keepdims=True))
    cbn = cb / (n2 + EPS)
    cb_ref[...] = cb
    cbn_ref[...] = cbn


def _normalize_cb(codebook):
    return pl.pallas_call(
        _normalize_cb_body,
        grid=(8,),
        in_specs=[pl.BlockSpec((N // 8, D), lambda i: (i, 0))],
        out_specs=[
            pl.BlockSpec((N // 8, D), lambda i: (i, 0)),
            pl.BlockSpec((N // 8, D), lambda i: (i, 0)),
        ],
        out_shape=[
            jax.ShapeDtypeStruct((N, D), jnp.float32),
            jax.ShapeDtypeStruct((N, D), jnp.float32),
        ],
    )(codebook)


def _normalize_x_body(x_ref, xn_ref):
    x = x_ref[...]
    nrm = jnp.sqrt(jnp.sum(x * x, axis=1, keepdims=True))
    xn_ref[...] = x / (nrm + EPS)


def _normalize_x(x):
    return pl.pallas_call(
        _normalize_x_body,
        grid=(16,),
        in_specs=[pl.BlockSpec((M // 16, D), lambda i: (i, 0))],
        out_specs=pl.BlockSpec((M // 16, D), lambda i: (i, 0)),
        out_shape=jax.ShapeDtypeStruct((M, D), jnp.float32),
    )(x)


def _argmax_body(xn_ref, cbn_ref, vmax_ref, vidx_ref):
    n = pl.program_id(1)

    @pl.when(n == 0)
    def _init():
        vmax_ref[...] = jnp.full((1, BM, LANES), -jnp.inf, jnp.float32)
        vidx_ref[...] = jnp.zeros((1, BM, LANES), jnp.int32)

    cbn = cbn_ref[...]
    xn = xn_ref[...]
    s = lax.dot_general(xn, cbn, (((1,), (1,)), ((), ())),
                        preferred_element_type=jnp.float32)
    for r in range(BM // RCH):
        rows = slice(r * RCH, (r + 1) * RCH)
        v = vmax_ref[0, rows, :]
        ii = vidx_ref[0, rows, :]
        for g in range(NG):
            sg = s[rows, g * LANES:(g + 1) * LANES]
            gt = sg > v
            v = jnp.maximum(sg, v)
            ii = jnp.where(gt, n * NG + g, ii)
        vmax_ref[0, rows, :] = v
        vidx_ref[0, rows, :] = ii


def _normalize_argmax(xn, cbn):
    vmax3, vidx3 = pl.pallas_call(
        _argmax_body,
        grid=(MB, NB),
        in_specs=[
            pl.BlockSpec((BM, D), lambda m, n: (m, 0)),
            pl.BlockSpec((BN, D), lambda m, n: (n, 0)),
        ],
        out_specs=[
            pl.BlockSpec((1, BM, LANES), lambda m, n: (m, 0, 0)),
            pl.BlockSpec((1, BM, LANES), lambda m, n: (m, 0, 0)),
        ],
        out_shape=[
            jax.ShapeDtypeStruct((MB, BM, LANES), jnp.float32),
            jax.ShapeDtypeStruct((MB, BM, LANES), jnp.int32),
        ],
    )(xn, cbn)
    return vmax3.reshape(M, LANES), vidx3.reshape(M, LANES)


FR = 2048


def _finish_body(v_ref, i_ref, idx_ref):
    v = v_ref[...]
    ii = i_ref[...]
    bmax = jnp.max(v, axis=1, keepdims=True)
    lane = lax.broadcasted_iota(jnp.int32, (FR, LANES), 1)
    idxfull = ii * LANES + lane
    bidx = jnp.min(jnp.where(v == bmax, idxfull, N), axis=1)
    idx_ref[...] = bidx.reshape(1, 1, FR)


def _finish(vmax, vidx):
    idx3 = pl.pallas_call(
        _finish_body,
        grid=(M // FR,),
        in_specs=[
            pl.BlockSpec((FR, LANES), lambda i: (i, 0)),
            pl.BlockSpec((FR, LANES), lambda i: (i, 0)),
        ],
        out_specs=pl.BlockSpec((1, 1, FR), lambda i: (i, 0, 0)),
        out_shape=jax.ShapeDtypeStruct((M // FR, 1, FR), jnp.int32),
    )(vmax, vidx)
    return idx3.reshape(M)


def _gather_body(cb_hbm, idx_hbm, z_hbm, idx_v, rows_v, sem):
    c = lax.axis_index("c")
    s = lax.axis_index("s")
    wid = s * NC + c
    base = wid * ROWS_PER_W
    for ch in range(ROWS_PER_W // GCHUNK):
        off = base + ch * GCHUNK
        pltpu.sync_copy(idx_hbm.at[pl.ds(off, GCHUNK)], idx_v)
        pltpu.async_copy(cb_hbm.at[idx_v], rows_v, sem).wait()
        pltpu.sync_copy(rows_v, z_hbm.at[pl.ds(off, GCHUNK)])


@functools.cache
def _sc_gather():
    return pl.kernel(
        _gather_body,
        out_type=jax.ShapeDtypeStruct((M, D), jnp.float32),
        mesh=plsc.VectorSubcoreMesh(core_axis_name="c", subcore_axis_name="s",
                                    num_cores=NC, num_subcores=NS),
        scratch_types=[
            pltpu.VMEM((GCHUNK,), jnp.int32),
            pltpu.VMEM((GCHUNK, D), jnp.float32),
            pltpu.SemaphoreType.DMA,
        ],
    )


def kernel(x, codebook, training=False):
    del training
    cb, cbn = _normalize_cb(codebook)
    xn = _normalize_x(x)
    vmax, vidx = _normalize_argmax(xn, cbn)
    indices = _finish(vmax, vidx)
    z = _sc_gather()(cb, indices)
    return (z, z, xn, indices)


# speedup vs baseline: 2.4466x; 1.7420x over previous
import functools

import jax
import jax.numpy as jnp
from jax import lax
from jax.experimental import pallas as pl
from jax.experimental.pallas import tpu as pltpu
from jax.experimental.pallas import tpu_sc as plsc

M = 16384
N = 8192
D = 256

BM = 2048
BN = 1024
MB = M // BM
NB = N // BN

EPS = 1e-8

LANES = 128
NG = BN // LANES
RCH = 64

NC = 2
NS = 16
NW = NC * NS
ROWS_PER_W = M // NW
GCHUNK = 256


def _normalize_cb_body(c_ref, cb_ref, cbn_ref):
    c = c_ref[...]
    n1 = jnp.sqrt(jnp.sum(c * c, axis=1, keepdims=True))
    cb = c / (n1 + EPS)
    n2 = jnp.sqrt(jnp.sum(cb * cb, axis=1, keepdims=True))
    cbn = cb / (n2 + EPS)
    cb_ref[...] = cb
    cbn_ref[...] = cbn


def _normalize_cb(codebook):
    return pl.pallas_call(
        _normalize_cb_body,
        grid=(8,),
        in_specs=[pl.BlockSpec((N // 8, D), lambda i: (i, 0))],
        out_specs=[
            pl.BlockSpec((N // 8, D), lambda i: (i, 0)),
            pl.BlockSpec((N // 8, D), lambda i: (i, 0)),
        ],
        out_shape=[
            jax.ShapeDtypeStruct((N, D), jnp.float32),
            jax.ShapeDtypeStruct((N, D), jnp.float32),
        ],
    )(codebook)


def _normalize_x_body(x_ref, xn_ref):
    x = x_ref[...]
    nrm = jnp.sqrt(jnp.sum(x * x, axis=1, keepdims=True))
    xn_ref[...] = x / (nrm + EPS)


def _normalize_x(x):
    return pl.pallas_call(
        _normalize_x_body,
        grid=(16,),
        in_specs=[pl.BlockSpec((M // 16, D), lambda i: (i, 0))],
        out_specs=pl.BlockSpec((M // 16, D), lambda i: (i, 0)),
        out_shape=jax.ShapeDtypeStruct((M, D), jnp.float32),
    )(x)


def _argmax_body(xn_ref, cbn_ref, vmax_ref, vidx_ref):
    n = pl.program_id(1)

    @pl.when(n == 0)
    def _init():
        vmax_ref[...] = jnp.full((1, BM, LANES), -jnp.inf, jnp.float32)
        vidx_ref[...] = jnp.zeros((1, BM, LANES), jnp.int32)

    cbn = cbn_ref[...]
    xn = xn_ref[...]
    s = lax.dot_general(xn, cbn, (((1,), (1,)), ((), ())),
                        preferred_element_type=jnp.float32)
    for r in range(BM // RCH):
        rows = slice(r * RCH, (r + 1) * RCH)
        v = vmax_ref[0, rows, :]
        ii = vidx_ref[0, rows, :]
        for g in range(NG):
            sg = s[rows, g * LANES:(g + 1) * LANES]
            gt = sg > v
            v = jnp.maximum(sg, v)
            ii = jnp.where(gt, n * NG + g, ii)
        vmax_ref[0, rows, :] = v
        vidx_ref[0, rows, :] = ii


def _normalize_argmax(xn, cbn):
    vmax3, vidx3 = pl.pallas_call(
        _argmax_body,
        grid=(MB, NB),
        in_specs=[
            pl.BlockSpec((BM, D), lambda m, n: (m, 0)),
            pl.BlockSpec((BN, D), lambda m, n: (n, 0)),
        ],
        out_specs=[
            pl.BlockSpec((1, BM, LANES), lambda m, n: (m, 0, 0)),
            pl.BlockSpec((1, BM, LANES), lambda m, n: (m, 0, 0)),
        ],
        out_shape=[
            jax.ShapeDtypeStruct((MB, BM, LANES), jnp.float32),
            jax.ShapeDtypeStruct((MB, BM, LANES), jnp.int32),
        ],
    )(xn, cbn)
    return vmax3.reshape(M, LANES), vidx3.reshape(M, LANES)


FR = 2048


def _finish_body(v_ref, i_ref, idx_ref):
    v = v_ref[...]
    ii = i_ref[...]
    bmax = jnp.max(v, axis=1, keepdims=True)
    lane = lax.broadcasted_iota(jnp.int32, (FR, LANES), 1)
    idxfull = ii * LANES + lane
    bidx = jnp.min(jnp.where(v == bmax, idxfull, N), axis=1)
    idx_ref[...] = bidx.reshape(1, 1, FR)


def _finish(vmax, vidx):
    idx3 = pl.pallas_call(
        _finish_body,
        grid=(M // FR,),
        in_specs=[
            pl.BlockSpec((FR, LANES), lambda i: (i, 0)),
            pl.BlockSpec((FR, LANES), lambda i: (i, 0)),
        ],
        out_specs=pl.BlockSpec((1, 1, FR), lambda i: (i, 0, 0)),
        out_shape=jax.ShapeDtypeStruct((M // FR, 1, FR), jnp.int32),
    )(vmax, vidx)
    return idx3.reshape(M)


def _gather_body(cb_hbm, idx_hbm, z_hbm, idx_v, rows_v, sem):
    c = lax.axis_index("c")
    s = lax.axis_index("s")
    wid = s * NC + c
    base = wid * ROWS_PER_W
    for ch in range(ROWS_PER_W // GCHUNK):
        off = base + ch * GCHUNK
        pltpu.sync_copy(idx_hbm.at[pl.ds(off, GCHUNK)], idx_v)
        pltpu.async_copy(cb_hbm.at[idx_v], rows_v, sem).wait()
        pltpu.sync_copy(rows_v, z_hbm.at[pl.ds(off, GCHUNK)])


@functools.cache
def _sc_gather():
    return pl.kernel(
        _gather_body,
        out_type=jax.ShapeDtypeStruct((M, D), jnp.float32),
        mesh=plsc.VectorSubcoreMesh(core_axis_name="c", subcore_axis_name="s",
                                    num_cores=NC, num_subcores=NS),
        scratch_types=[
            pltpu.VMEM((GCHUNK,), jnp.int32),
            pltpu.VMEM((GCHUNK, D), jnp.float32),
            pltpu.SemaphoreType.DMA,
        ],
    )


def kernel(x, codebook, training=False):
    del training
    cb, cbn = _normalize_cb(codebook)
    xn = _normalize_x(x)
    vmax, vidx = _normalize_argmax(xn, cbn)
    indices = _finish(vmax, vidx)
    z = _sc_gather()(cb, indices)
    return (z, z, xn, indices)


# speedup vs baseline: 2.6392x; 1.0787x over previous
import functools

import jax
import jax.numpy as jnp
from jax import lax
from jax.experimental import pallas as pl
from jax.experimental.pallas import tpu as pltpu
from jax.experimental.pallas import tpu_sc as plsc

M = 16384
N = 8192
D = 256

BM = 4096
BN = 1024
MB = M // BM
NB = N // BN

EPS = 1e-8

LANES = 128
NG = BN // LANES
RCH = 64

NC = 2
NS = 16
NW = NC * NS
ROWS_PER_W = M // NW
GCHUNK = 256


def _normalize_cb_body(c_ref, cb_ref, cbn_ref):
    c = c_ref[...]
    n1 = jnp.sqrt(jnp.sum(c * c, axis=1, keepdims=True))
    cb = c / (n1 + EPS)
    n2 = jnp.sqrt(jnp.sum(cb * cb, axis=1, keepdims=True))
    cbn = cb / (n2 + EPS)
    cb_ref[...] = cb
    cbn_ref[...] = cbn


def _normalize_cb(codebook):
    return pl.pallas_call(
        _normalize_cb_body,
        grid=(8,),
        in_specs=[pl.BlockSpec((N // 8, D), lambda i: (i, 0))],
        out_specs=[
            pl.BlockSpec((N // 8, D), lambda i: (i, 0)),
            pl.BlockSpec((N // 8, D), lambda i: (i, 0)),
        ],
        out_shape=[
            jax.ShapeDtypeStruct((N, D), jnp.float32),
            jax.ShapeDtypeStruct((N, D), jnp.float32),
        ],
    )(codebook)


def _normalize_x_body(x_ref, xn_ref):
    x = x_ref[...]
    nrm = jnp.sqrt(jnp.sum(x * x, axis=1, keepdims=True))
    xn_ref[...] = x / (nrm + EPS)


def _normalize_x(x):
    return pl.pallas_call(
        _normalize_x_body,
        grid=(16,),
        in_specs=[pl.BlockSpec((M // 16, D), lambda i: (i, 0))],
        out_specs=pl.BlockSpec((M // 16, D), lambda i: (i, 0)),
        out_shape=jax.ShapeDtypeStruct((M, D), jnp.float32),
    )(x)


def _argmax_body(xn_ref, cbn_ref, vmax_ref, vidx_ref):
    n = pl.program_id(1)

    @pl.when(n == 0)
    def _init():
        vmax_ref[...] = jnp.full((1, BM, LANES), -jnp.inf, jnp.float32)
        vidx_ref[...] = jnp.zeros((1, BM, LANES), jnp.int32)

    cbn = cbn_ref[...]
    xn = xn_ref[...]
    s = lax.dot_general(xn, cbn, (((1,), (1,)), ((), ())),
                        preferred_element_type=jnp.float32)
    for r in range(BM // RCH):
        rows = slice(r * RCH, (r + 1) * RCH)
        v = vmax_ref[0, rows, :]
        ii = vidx_ref[0, rows, :]
        for g in range(NG):
            sg = s[rows, g * LANES:(g + 1) * LANES]
            gt = sg > v
            v = jnp.maximum(sg, v)
            ii = jnp.where(gt, n * NG + g, ii)
        vmax_ref[0, rows, :] = v
        vidx_ref[0, rows, :] = ii


def _normalize_argmax(xn, cbn):
    vmax3, vidx3 = pl.pallas_call(
        _argmax_body,
        grid=(MB, NB),
        in_specs=[
            pl.BlockSpec((BM, D), lambda m, n: (m, 0)),
            pl.BlockSpec((BN, D), lambda m, n: (n, 0)),
        ],
        out_specs=[
            pl.BlockSpec((1, BM, LANES), lambda m, n: (m, 0, 0)),
            pl.BlockSpec((1, BM, LANES), lambda m, n: (m, 0, 0)),
        ],
        out_shape=[
            jax.ShapeDtypeStruct((MB, BM, LANES), jnp.float32),
            jax.ShapeDtypeStruct((MB, BM, LANES), jnp.int32),
        ],
    )(xn, cbn)
    return vmax3.reshape(M, LANES), vidx3.reshape(M, LANES)


FR = 2048


def _finish_body(v_ref, i_ref, idx_ref):
    v = v_ref[...]
    ii = i_ref[...]
    bmax = jnp.max(v, axis=1, keepdims=True)
    lane = lax.broadcasted_iota(jnp.int32, (FR, LANES), 1)
    idxfull = ii * LANES + lane
    bidx = jnp.min(jnp.where(v == bmax, idxfull, N), axis=1)
    idx_ref[...] = bidx.reshape(1, 1, FR)


def _finish(vmax, vidx):
    idx3 = pl.pallas_call(
        _finish_body,
        grid=(M // FR,),
        in_specs=[
            pl.BlockSpec((FR, LANES), lambda i: (i, 0)),
            pl.BlockSpec((FR, LANES), lambda i: (i, 0)),
        ],
        out_specs=pl.BlockSpec((1, 1, FR), lambda i: (i, 0, 0)),
        out_shape=jax.ShapeDtypeStruct((M // FR, 1, FR), jnp.int32),
    )(vmax, vidx)
    return idx3.reshape(M)


def _gather_body(cb_hbm, idx_hbm, z_hbm, idx_v, rows_v, sem):
    c = lax.axis_index("c")
    s = lax.axis_index("s")
    wid = s * NC + c
    base = wid * ROWS_PER_W
    for ch in range(ROWS_PER_W // GCHUNK):
        off = base + ch * GCHUNK
        pltpu.sync_copy(idx_hbm.at[pl.ds(off, GCHUNK)], idx_v)
        pltpu.async_copy(cb_hbm.at[idx_v], rows_v, sem).wait()
        pltpu.sync_copy(rows_v, z_hbm.at[pl.ds(off, GCHUNK)])


@functools.cache
def _sc_gather():
    return pl.kernel(
        _gather_body,
        out_type=jax.ShapeDtypeStruct((M, D), jnp.float32),
        mesh=plsc.VectorSubcoreMesh(core_axis_name="c", subcore_axis_name="s",
                                    num_cores=NC, num_subcores=NS),
        scratch_types=[
            pltpu.VMEM((GCHUNK,), jnp.int32),
            pltpu.VMEM((GCHUNK, D), jnp.float32),
            pltpu.SemaphoreType.DMA,
        ],
    )


def kernel(x, codebook, training=False):
    del training
    cb, cbn = _normalize_cb(codebook)
    xn = _normalize_x(x)
    vmax, vidx = _normalize_argmax(xn, cbn)
    indices = _finish(vmax, vidx)
    z = _sc_gather()(cb, indices)
    return (z, z, xn, indices)


# speedup vs baseline: 2.6772x; 1.0144x over previous
import functools

import jax
import jax.numpy as jnp
from jax import lax
from jax.experimental import pallas as pl
from jax.experimental.pallas import tpu as pltpu
from jax.experimental.pallas import tpu_sc as plsc

M = 16384
N = 8192
D = 256

BM = 8192
BN = 1024
MB = M // BM
NB = N // BN

EPS = 1e-8

LANES = 128
NG = BN // LANES
RCH = 64

NC = 2
NS = 16
NW = NC * NS
ROWS_PER_W = M // NW
GCHUNK = 256


def _normalize_cb_body(c_ref, cb_ref, cbn_ref):
    c = c_ref[...]
    n1 = jnp.sqrt(jnp.sum(c * c, axis=1, keepdims=True))
    cb = c / (n1 + EPS)
    n2 = jnp.sqrt(jnp.sum(cb * cb, axis=1, keepdims=True))
    cbn = cb / (n2 + EPS)
    cb_ref[...] = cb
    cbn_ref[...] = cbn


def _normalize_cb(codebook):
    return pl.pallas_call(
        _normalize_cb_body,
        grid=(8,),
        in_specs=[pl.BlockSpec((N // 8, D), lambda i: (i, 0))],
        out_specs=[
            pl.BlockSpec((N // 8, D), lambda i: (i, 0)),
            pl.BlockSpec((N // 8, D), lambda i: (i, 0)),
        ],
        out_shape=[
            jax.ShapeDtypeStruct((N, D), jnp.float32),
            jax.ShapeDtypeStruct((N, D), jnp.float32),
        ],
    )(codebook)


def _normalize_x_body(x_ref, xn_ref):
    x = x_ref[...]
    nrm = jnp.sqrt(jnp.sum(x * x, axis=1, keepdims=True))
    xn_ref[...] = x / (nrm + EPS)


def _normalize_x(x):
    return pl.pallas_call(
        _normalize_x_body,
        grid=(16,),
        in_specs=[pl.BlockSpec((M // 16, D), lambda i: (i, 0))],
        out_specs=pl.BlockSpec((M // 16, D), lambda i: (i, 0)),
        out_shape=jax.ShapeDtypeStruct((M, D), jnp.float32),
    )(x)


def _argmax_body(xn_ref, cbn_ref, vmax_ref, vidx_ref):
    n = pl.program_id(1)

    @pl.when(n == 0)
    def _init():
        vmax_ref[...] = jnp.full((1, BM, LANES), -jnp.inf, jnp.float32)
        vidx_ref[...] = jnp.zeros((1, BM, LANES), jnp.int32)

    cbn = cbn_ref[...]
    xn = xn_ref[...]
    s = lax.dot_general(xn, cbn, (((1,), (1,)), ((), ())),
                        preferred_element_type=jnp.float32)
    for r in range(BM // RCH):
        rows = slice(r * RCH, (r + 1) * RCH)
        v = vmax_ref[0, rows, :]
        ii = vidx_ref[0, rows, :]
        for g in range(NG):
            sg = s[rows, g * LANES:(g + 1) * LANES]
            gt = sg > v
            v = jnp.maximum(sg, v)
            ii = jnp.where(gt, n * NG + g, ii)
        vmax_ref[0, rows, :] = v
        vidx_ref[0, rows, :] = ii


def _normalize_argmax(xn, cbn):
    vmax3, vidx3 = pl.pallas_call(
        _argmax_body,
        grid=(MB, NB),
        in_specs=[
            pl.BlockSpec((BM, D), lambda m, n: (m, 0)),
            pl.BlockSpec((BN, D), lambda m, n: (n, 0)),
        ],
        out_specs=[
            pl.BlockSpec((1, BM, LANES), lambda m, n: (m, 0, 0)),
            pl.BlockSpec((1, BM, LANES), lambda m, n: (m, 0, 0)),
        ],
        out_shape=[
            jax.ShapeDtypeStruct((MB, BM, LANES), jnp.float32),
            jax.ShapeDtypeStruct((MB, BM, LANES), jnp.int32),
        ],
    )(xn, cbn)
    return vmax3.reshape(M, LANES), vidx3.reshape(M, LANES)


FR = 2048


def _finish_body(v_ref, i_ref, idx_ref):
    v = v_ref[...]
    ii = i_ref[...]
    bmax = jnp.max(v, axis=1, keepdims=True)
    lane = lax.broadcasted_iota(jnp.int32, (FR, LANES), 1)
    idxfull = ii * LANES + lane
    bidx = jnp.min(jnp.where(v == bmax, idxfull, N), axis=1)
    idx_ref[...] = bidx.reshape(1, 1, FR)


def _finish(vmax, vidx):
    idx3 = pl.pallas_call(
        _finish_body,
        grid=(M // FR,),
        in_specs=[
            pl.BlockSpec((FR, LANES), lambda i: (i, 0)),
            pl.BlockSpec((FR, LANES), lambda i: (i, 0)),
        ],
        out_specs=pl.BlockSpec((1, 1, FR), lambda i: (i, 0, 0)),
        out_shape=jax.ShapeDtypeStruct((M // FR, 1, FR), jnp.int32),
    )(vmax, vidx)
    return idx3.reshape(M)


def _gather_body(cb_hbm, idx_hbm, z_hbm, idx_v, rows_v, sem):
    c = lax.axis_index("c")
    s = lax.axis_index("s")
    wid = s * NC + c
    base = wid * ROWS_PER_W
    for ch in range(ROWS_PER_W // GCHUNK):
        off = base + ch * GCHUNK
        pltpu.sync_copy(idx_hbm.at[pl.ds(off, GCHUNK)], idx_v)
        pltpu.async_copy(cb_hbm.at[idx_v], rows_v, sem).wait()
        pltpu.sync_copy(rows_v, z_hbm.at[pl.ds(off, GCHUNK)])


@functools.cache
def _sc_gather():
    return pl.kernel(
        _gather_body,
        out_type=jax.ShapeDtypeStruct((M, D), jnp.float32),
        mesh=plsc.VectorSubcoreMesh(core_axis_name="c", subcore_axis_name="s",
                                    num_cores=NC, num_subcores=NS),
        scratch_types=[
            pltpu.VMEM((GCHUNK,), jnp.int32),
            pltpu.VMEM((GCHUNK, D), jnp.float32),
            pltpu.SemaphoreType.DMA,
        ],
    )


def kernel(x, codebook, training=False):
    del training
    cb, cbn = _normalize_cb(codebook)
    xn = _normalize_x(x)
    vmax, vidx = _normalize_argmax(xn, cbn)
    indices = _finish(vmax, vidx)
    z = _sc_gather()(cb, indices)
    return (z, z, xn, indices)
